# Initial kernel scaffold; baseline (speedup 1.0000x reference)
import jax
import jax.numpy as jnp
from jax.experimental import pallas as pl


def kernel(node_feature, edge_index, node2graph, W, b, query):
    raise NotImplementedError("write your pallas kernel here")


# speedup vs baseline: 10.5863x; 10.5863x over previous
import dataclasses
import functools

import jax
import jax.numpy as jnp
from jax import lax
from jax.experimental import pallas as pl
from jax.experimental.pallas import tpu as pltpu
from jax.experimental.pallas import tpu_sc as plsc

H = 4
D = 128
DH = D // H
N = 10000
E = 320000
G = 64
EPS = 1e-10
NEG = 0.2

NC = 2
NS = 16
NW = NC * NS
L = 16
EPT = E // NW
K = 80
NB = 10
BN = N // NB
RPT = 624
NPR = 25
ER = 400


def _prep_body(x_ref, wt_ref, b_ref, q_ref, hid_ref, a_ref):
    hid = jnp.dot(x_ref[...], wt_ref[...], preferred_element_type=jnp.float32)
    hid = hid + b_ref[...]
    hid_ref[...] = hid
    a_ref[...] = jnp.dot(hid, q_ref[...], preferred_element_type=jnp.float32)


@jax.jit
def _prep(x, wt, b2, q):
    return pl.pallas_call(
        _prep_body,
        grid=(NB,),
        in_specs=[
            pl.BlockSpec((BN, D), lambda i: (i, 0)),
            pl.BlockSpec((D, D), lambda i: (0, 0)),
            pl.BlockSpec((1, D), lambda i: (0, 0)),
            pl.BlockSpec((D, 2 * H), lambda i: (0, 0)),
        ],
        out_specs=[
            pl.BlockSpec((BN, D), lambda i: (i, 0)),
            pl.BlockSpec((BN, 2 * H), lambda i: (i, 0)),
        ],
        out_shape=[
            jax.ShapeDtypeStruct((N, D), jnp.float32),
            jax.ShapeDtypeStruct((N, 2 * H), jnp.float32),
        ],
    )(x, wt, b2, q)


def _att_body(a_hbm, src_hbm, dst_hbm, e_hbm, oute_hbm,
              atab, sidx, didx, didx2, ebuf, erows, acce):
    cid = lax.axis_index("c")
    sid = lax.axis_index("s")
    wid = cid * NS + sid

    pltpu.sync_copy(a_hbm, atab)

    zrow = jnp.zeros((L,), jnp.float32)

    @pl.loop(0, K)
    def _(i):
        for c in range(D // L):
            erows[i, pl.ds(c * L, L)] = zrow

    @pl.when(sid == 0)
    def _():
        for c in range(ER // K):
            pltpu.sync_copy(erows, acce.at[pl.ds(c * K, K)])

    plsc.subcore_barrier()

    base = wid * EPT
    ones16 = jnp.ones((L,), jnp.float32)

    @pl.loop(0, EPT, step=K)
    def _(i0):
        pltpu.sync_copy(src_hbm.at[pl.ds(base + i0, K)], sidx)
        pltpu.sync_copy(dst_hbm.at[pl.ds(base + i0, K)], didx)

        for g in range(K // L):
            s16 = sidx[pl.ds(g * L, L)]
            d16 = didx[pl.ds(g * L, L)]
            ridx = lax.iota(jnp.int32, L) + g * L
            didx2[pl.ds(g * L, L)] = d16 // NPR
            ecol = (d16 % NPR) * 5
            s8 = s16 * (2 * H)
            d8 = d16 * (2 * H)
            plsc.store_scatter(erows, [ridx, ecol + 4], ones16)
            for h in range(H):
                sa = plsc.load_gather(atab, [s8 + h])
                da = plsc.load_gather(atab, [d8 + (H + h)])
                w = sa + da
                w = jnp.where(w >= 0.0, w, NEG * w)
                eh = jnp.exp(w)
                plsc.store_scatter(erows, [ridx, ecol + h], eh)
                plsc.store_scatter(ebuf, [ridx * H + h], eh)

        pltpu.sync_copy(ebuf, e_hbm.at[pl.ds((base + i0) * H, K * H)])
        pltpu.sync_copy(erows, acce.at[didx2], add=True)

        for g in range(K // L):
            d16 = didx[pl.ds(g * L, L)]
            ridx = lax.iota(jnp.int32, L) + g * L
            ecol = (d16 % NPR) * 5
            for s in range(5):
                plsc.store_scatter(erows, [ridx, ecol + s], zrow)

    plsc.subcore_barrier()

    @pl.when(sid == 0)
    def _():
        pltpu.sync_copy(acce, oute_hbm.at[cid])


@jax.jit
def _att(a_flat, src, dst):
    mesh = plsc.VectorSubcoreMesh(
        core_axis_name="c", subcore_axis_name="s",
        num_cores=NC, num_subcores=NS)
    cp = pltpu.CompilerParams()
    if "needs_layout_passes" in pltpu.CompilerParams.__dataclass_fields__:
        cp = dataclasses.replace(cp, needs_layout_passes=False)
    f = pl.kernel(
        _att_body,
        out_type=[
            jax.ShapeDtypeStruct((E * H,), jnp.float32),
            jax.ShapeDtypeStruct((NC, ER, D), jnp.float32),
        ],
        mesh=mesh,
        scratch_types=[
            pltpu.VMEM((N * 2 * H,), jnp.float32),
            pltpu.VMEM((K,), jnp.int32),
            pltpu.VMEM((K,), jnp.int32),
            pltpu.VMEM((K,), jnp.int32),
            pltpu.VMEM((K * H,), jnp.float32),
            pltpu.VMEM((K, D), jnp.float32),
            pltpu.VMEM_SHARED((ER, D), jnp.float32),
        ],
        compiler_params=cp,
    )
    return f(a_flat, src, dst)


def _msg_body(hid_hbm, e_hbm, src_hbm, dst_hbm, outm_hbm,
              sidx, didx, ebuf, rows, orows, accm, sem):
    cid = lax.axis_index("c")
    sid = lax.axis_index("s")
    wid = cid * NS + sid

    zrow = jnp.zeros((L,), jnp.float32)

    @pl.loop(0, K)
    def _(i):
        for c in range(D // L):
            orows[i, pl.ds(c * L, L)] = zrow

    for c in range(RPT // K):
        pltpu.sync_copy(orows, accm.at[pl.ds(sid * RPT + c * K, K)])
    rem = RPT - (RPT // K) * K
    if rem:
        pltpu.sync_copy(orows.at[pl.ds(0, rem)],
                        accm.at[pl.ds(sid * RPT + (RPT // K) * K, rem)])

    @pl.when(sid == NS - 1)
    def _():
        pltpu.sync_copy(orows.at[pl.ds(0, N - NS * RPT)],
                        accm.at[pl.ds(NS * RPT, N - NS * RPT)])

    plsc.subcore_barrier()

    base = wid * EPT

    @pl.loop(0, EPT, step=K)
    def _(i0):
        pltpu.sync_copy(src_hbm.at[pl.ds(base + i0, K)], sidx)
        pltpu.sync_copy(dst_hbm.at[pl.ds(base + i0, K)], didx)
        pltpu.sync_copy(e_hbm.at[pl.ds((base + i0) * H, K * H)], ebuf)
        pltpu.async_copy(hid_hbm.at[sidx], rows, sem).wait()

        for q in range(K // H):
            e16 = ebuf[pl.ds(q * L, L)]
            for j in range(H):
                k = q * H + j
                for h in range(H):
                    ej = lax.gather(
                        e16, jnp.full((L, 1), j * H + h, jnp.int32),
                        lax.GatherDimensionNumbers(
                            offset_dims=(), collapsed_slice_dims=(0,),
                            start_index_map=(0,)),
                        slice_sizes=(1,),
                        mode=lax.GatherScatterMode.PROMISE_IN_BOUNDS)
                    orows[k, pl.ds(h * DH, L)] = rows[k, pl.ds(h * DH, L)] * ej
                    orows[k, pl.ds(h * DH + L, L)] = rows[k, pl.ds(h * DH + L, L)] * ej

        pltpu.sync_copy(orows, accm.at[didx], add=True)

    plsc.subcore_barrier()

    pltpu.sync_copy(accm.at[pl.ds(sid * RPT, RPT)],
                    outm_hbm.at[cid, pl.ds(sid * RPT, RPT)])

    @pl.when(sid == NS - 1)
    def _():
        pltpu.sync_copy(accm.at[pl.ds(NS * RPT, N - NS * RPT)],
                        outm_hbm.at[cid, pl.ds(NS * RPT, N - NS * RPT)])


@jax.jit
def _msg(hidden, e_all, src, dst):
    mesh = plsc.VectorSubcoreMesh(
        core_axis_name="c", subcore_axis_name="s",
        num_cores=NC, num_subcores=NS)
    cp = pltpu.CompilerParams()
    if "needs_layout_passes" in pltpu.CompilerParams.__dataclass_fields__:
        cp = dataclasses.replace(cp, needs_layout_passes=False)
    f = pl.kernel(
        _msg_body,
        out_type=jax.ShapeDtypeStruct((NC, N, D), jnp.float32),
        mesh=mesh,
        scratch_types=[
            pltpu.VMEM((K,), jnp.int32),
            pltpu.VMEM((K,), jnp.int32),
            pltpu.VMEM((K * H,), jnp.float32),
            pltpu.VMEM((K, D), jnp.float32),
            pltpu.VMEM((K, D), jnp.float32),
            pltpu.VMEM_SHARED((N, D), jnp.float32),
            pltpu.SemaphoreType.DMA,
        ],
        compiler_params=cp,
    )
    return f(hidden, e_all, src, dst)


def _finish_body(pa_ref, pb_ref, ea_ref, eb_ref, hid_ref, a_ref, n2g_ref,
                 nf_ref, hg_ref, gsum, gcnt):
    i = pl.program_id(0)
    msg = pa_ref[...] + pb_ref[...]
    ep = ea_ref[...] + eb_ref[...]
    esum = ep[:, :H]
    hid = hid_ref[...]
    a = a_ref[...]

    ws = a[:, :H] + a[:, H:]
    ws = jnp.where(ws >= 0.0, ws, NEG * ws)
    es = jnp.exp(ws)

    cnt = ep[:, H:H + 1] + 1.0

    parts = []
    for h in range(H):
        msg_h = (msg[:, h * DH:(h + 1) * DH]
                 + es[:, h:h + 1] * hid[:, h * DH:(h + 1) * DH])
        den_h = esum[:, h:h + 1] + es[:, h:h + 1] + EPS * cnt
        parts.append(msg_h / den_h)
    nf = jnp.maximum(jnp.concatenate(parts, axis=1), 0.0)
    nf_ref[...] = nf

    n2g = n2g_ref[...]
    iota = lax.broadcasted_iota(jnp.int32, (BN, G), 1).astype(jnp.float32)
    oh = (n2g == iota).astype(jnp.float32)
    dn = (((0,), (0,)), ((), ()))
    blk_sum = lax.dot_general(oh, nf, dn, preferred_element_type=jnp.float32)
    blk_cnt = lax.dot_general(oh, jnp.ones((BN, 1), jnp.float32), dn,
                              preferred_element_type=jnp.float32)

    @pl.when(i == 0)
    def _():
        gsum[...] = jnp.zeros_like(gsum)
        gcnt[...] = jnp.zeros_like(gcnt)

    gsum[...] += blk_sum
    gcnt[...] += blk_cnt

    @pl.when(i == NB - 1)
    def _():
        hg_ref[...] = gsum[...] / jnp.maximum(gcnt[...], 1.0)


@jax.jit
def _finish(pa, pb, ea, eb, hidden, a, n2g_f):
    return pl.pallas_call(
        _finish_body,
        grid=(NB,),
        in_specs=[
            pl.BlockSpec((BN, D), lambda i: (i, 0)),
            pl.BlockSpec((BN, D), lambda i: (i, 0)),
            pl.BlockSpec((BN, 5), lambda i: (i, 0)),
            pl.BlockSpec((BN, 5), lambda i: (i, 0)),
            pl.BlockSpec((BN, D), lambda i: (i, 0)),
            pl.BlockSpec((BN, 2 * H), lambda i: (i, 0)),
            pl.BlockSpec((BN, 1), lambda i: (i, 0)),
        ],
        out_specs=[
            pl.BlockSpec((BN, D), lambda i: (i, 0)),
            pl.BlockSpec((G, D), lambda i: (0, 0)),
        ],
        out_shape=[
            jax.ShapeDtypeStruct((N, D), jnp.float32),
            jax.ShapeDtypeStruct((G, D), jnp.float32),
        ],
        scratch_shapes=[
            pltpu.VMEM((G, D), jnp.float32),
            pltpu.VMEM((G, 1), jnp.float32),
        ],
    )(pa, pb, ea, eb, hidden, a, n2g_f)


def kernel(node_feature, edge_index, node2graph, W, b, query):
    qi = query[:, 0::2]
    qo = query[:, 1::2]
    Q = jnp.zeros((D, 2 * H), jnp.float32)
    for h in range(H):
        Q = Q.at[h * DH:(h + 1) * DH, h].set(qi[h])
        Q = Q.at[h * DH:(h + 1) * DH, H + h].set(qo[h])

    hidden, a = _prep(node_feature, W.T, b.reshape(1, D), Q)
    src, dst = edge_index[0], edge_index[1]
    e_all, e2 = _att(a.reshape(-1), src, dst)
    msg2 = _msg(hidden, e_all, src, dst)
    ea = e2[0][:, :NPR * 5].reshape(N, 5)
    eb = e2[1][:, :NPR * 5].reshape(N, 5)
    n2g_f = node2graph.astype(jnp.float32).reshape(N, 1)
    new_nf, hg = _finish(msg2[0], msg2[1], ea, eb, hidden, a, n2g_f)
    return new_nf, hg


# speedup vs baseline: 25.4546x; 2.4045x over previous
import dataclasses
import functools

import jax
import jax.numpy as jnp
from jax import lax
from jax.experimental import pallas as pl
from jax.experimental.pallas import tpu as pltpu
from jax.experimental.pallas import tpu_sc as plsc

H = 4
D = 128
DH = D // H
N = 10000
E = 320000
G = 64
EPS = 1e-10
NEG = 0.2

NC = 2
NS = 16
NW = NC * NS
L = 16
EPT = E // NW
K = 80
NB = 10
BN = N // NB
RPT = 624
NPR = 32
ER = 320


def _prep_body(x_ref, wt_ref, b_ref, q_ref, hid_ref, a_ref):
    hid = jnp.dot(x_ref[...], wt_ref[...], preferred_element_type=jnp.float32)
    hid = hid + b_ref[...]
    hid_ref[...] = hid
    a_ref[...] = jnp.dot(hid, q_ref[...], preferred_element_type=jnp.float32)


@jax.jit
def _prep(x, wt, b2, q):
    return pl.pallas_call(
        _prep_body,
        grid=(NB,),
        in_specs=[
            pl.BlockSpec((BN, D), lambda i: (i, 0)),
            pl.BlockSpec((D, D), lambda i: (0, 0)),
            pl.BlockSpec((1, D), lambda i: (0, 0)),
            pl.BlockSpec((D, 2 * H), lambda i: (0, 0)),
        ],
        out_specs=[
            pl.BlockSpec((BN, D), lambda i: (i, 0)),
            pl.BlockSpec((BN, 2 * H), lambda i: (i, 0)),
        ],
        out_shape=[
            jax.ShapeDtypeStruct((N, D), jnp.float32),
            jax.ShapeDtypeStruct((N, 2 * H), jnp.float32),
        ],
    )(x, wt, b2, q)


def _att_body(a_hbm, src_hbm, dst_hbm, e_hbm, oute_hbm,
              atab, sidx, didx, didx2, ecolb, ebuf, erows, acce,
              semi, semw, seme):
    cid = lax.axis_index("c")
    sid = lax.axis_index("s")
    wid = cid * NS + sid
    base = wid * EPT

    pltpu.sync_copy(a_hbm, atab)

    zrow = jnp.zeros((L,), jnp.float32)
    ones16 = jnp.ones((L,), jnp.float32)

    @pl.loop(0, K)
    def _(i):
        for c in range(D // L):
            erows[0, i, pl.ds(c * L, L)] = zrow
            erows[1, i, pl.ds(c * L, L)] = zrow

    @pl.when(sid == 0)
    def _():
        for c in range(ER // K):
            pltpu.sync_copy(erows.at[0], acce.at[pl.ds(c * K, K)])

    plsc.subcore_barrier()

    def issue_idx(cexpr, b):
        off = base + cexpr * K
        pltpu.async_copy(src_hbm.at[pl.ds(off, K)], sidx.at[b, 0], semi.at[b])
        pltpu.async_copy(dst_hbm.at[pl.ds(off, K)], didx.at[b, 0], semi.at[b])

    def wait_idx(b):
        pltpu.make_async_copy(src_hbm.at[pl.ds(0, K)], sidx.at[b, 0],
                              semi.at[b]).wait()
        pltpu.make_async_copy(dst_hbm.at[pl.ds(0, K)], didx.at[b, 0],
                              semi.at[b]).wait()

    def zero_slots(rb):
        for g in range(K // L):
            ridx = lax.iota(jnp.int32, L) + g * L
            ecol = ecolb[rb, 0, pl.ds(g * L, L)]
            for s in range(H):
                plsc.store_scatter(erows.at[rb], [ridx, ecol + s], zrow)

    def compute(rb):
        for g in range(K // L):
            s16 = sidx[rb, 0, pl.ds(g * L, L)]
            d16 = didx[rb, 0, pl.ds(g * L, L)]
            ridx = lax.iota(jnp.int32, L) + g * L
            didx2[rb, 0, pl.ds(g * L, L)] = lax.shift_right_logical(d16, 5)
            ecol = (d16 & (NPR - 1)) * H
            ecolb[rb, 0, pl.ds(g * L, L)] = ecol
            s8 = s16 * (2 * H)
            d8 = d16 * (2 * H)
            for h in range(H):
                sa = plsc.load_gather(atab, [s8 + h])
                da = plsc.load_gather(atab, [d8 + (H + h)])
                w = sa + da
                w = jnp.where(w >= 0.0, w, NEG * w)
                eh = jnp.exp(w)
                plsc.store_scatter(erows.at[rb], [ridx, ecol + h], eh)
                plsc.store_scatter(ebuf.at[rb, 0], [ridx * H + h], eh)

    def issue_out(cexpr, rb):
        off = (base + cexpr * K) * H
        pltpu.async_copy(ebuf.at[rb, 0], e_hbm.at[pl.ds(off, K * H)],
                         semw.at[rb])
        pltpu.async_copy(erows.at[rb], acce.at[didx2.at[rb, 0]], seme.at[rb],
                         add=True)

    def wait_out(rb):
        pltpu.make_async_copy(ebuf.at[rb, 0], e_hbm.at[pl.ds(0, K * H)],
                              semw.at[rb]).wait()
        pltpu.make_async_copy(erows.at[rb], acce.at[didx2.at[rb, 0]],
                              seme.at[rb]).wait()

    issue_idx(0, 0)
    wait_idx(0)
    issue_idx(1, 1)

    NCH = EPT // K

    @pl.loop(0, NCH - 3, step=2)
    def _(c0):
        for j in range(2):
            c = c0 + j
            rb = j
            if j == 0:
                @pl.when(c0 > 0)
                def _():
                    wait_idx(0)
            else:
                wait_idx(1)

            @pl.when(c0 > 0)
            def _():
                wait_out(rb)
                zero_slots(rb)

            compute(rb)
            issue_out(c, rb)
            issue_idx(c + 2, rb)

    for c in (NCH - 3, NCH - 2, NCH - 1):
        rb = c % 2
        wait_idx(rb)
        wait_out(rb)
        zero_slots(rb)
        compute(rb)
        issue_out(c, rb)
        if c + 2 <= NCH - 1:
            issue_idx(c + 2, rb)
    wait_out((NCH - 2) % 2)
    wait_out((NCH - 1) % 2)

    plsc.subcore_barrier()

    @pl.when(sid == 0)
    def _():
        pltpu.sync_copy(acce, oute_hbm.at[cid])


@jax.jit
def _att(a_flat, src, dst):
    mesh = plsc.VectorSubcoreMesh(
        core_axis_name="c", subcore_axis_name="s",
        num_cores=NC, num_subcores=NS)
    cp = pltpu.CompilerParams()
    if "needs_layout_passes" in pltpu.CompilerParams.__dataclass_fields__:
        cp = dataclasses.replace(cp, needs_layout_passes=False)
    f = pl.kernel(
        _att_body,
        out_type=[
            jax.ShapeDtypeStruct((E * H,), jnp.float32),
            jax.ShapeDtypeStruct((NC, ER, D), jnp.float32),
        ],
        mesh=mesh,
        scratch_types=[
            pltpu.VMEM((N * 2 * H,), jnp.float32),
            pltpu.VMEM((2, 1, K), jnp.int32),
            pltpu.VMEM((2, 1, K), jnp.int32),
            pltpu.VMEM((2, 1, K), jnp.int32),
            pltpu.VMEM((2, 1, K), jnp.int32),
            pltpu.VMEM((2, 1, K * H), jnp.float32),
            pltpu.VMEM((2, K, D), jnp.float32),
            pltpu.VMEM_SHARED((ER, D), jnp.float32),
            pltpu.SemaphoreType.DMA((2,)),
            pltpu.SemaphoreType.DMA((2,)),
            pltpu.SemaphoreType.DMA((2,)),
        ],
        compiler_params=cp,
    )
    return f(a_flat, src, dst)


def _msg_body(hid_hbm, e_hbm, src_hbm, dst_hbm, outm_hbm,
              sidx, didx, didx_s, ebuf, rows, orows, accm,
              semi, semg, sems):
    cid = lax.axis_index("c")
    sid = lax.axis_index("s")
    wid = cid * NS + sid
    base = wid * EPT

    zrow = jnp.zeros((L,), jnp.float32)

    @pl.loop(0, K)
    def _(i):
        for c in range(D // L):
            orows[0, i, pl.ds(c * L, L)] = zrow
            orows[1, i, pl.ds(c * L, L)] = zrow

    for c in range(RPT // K):
        pltpu.sync_copy(orows.at[0], accm.at[pl.ds(sid * RPT + c * K, K)])
    rem = RPT - (RPT // K) * K
    if rem:
        pltpu.sync_copy(orows.at[0].at[pl.ds(0, rem)],
                        accm.at[pl.ds(sid * RPT + (RPT // K) * K, rem)])

    @pl.when(sid == NS - 1)
    def _():
        pltpu.sync_copy(orows.at[0].at[pl.ds(0, N - NS * RPT)],
                        accm.at[pl.ds(NS * RPT, N - NS * RPT)])

    plsc.subcore_barrier()

    def issue_idx(cexpr, b):
        off = base + cexpr * K
        pltpu.async_copy(src_hbm.at[pl.ds(off, K)], sidx.at[b, 0], semi.at[b])
        pltpu.async_copy(dst_hbm.at[pl.ds(off, K)], didx.at[b, 0], semi.at[b])
        pltpu.async_copy(e_hbm.at[pl.ds(off * H, K * H)], ebuf.at[b, 0],
                         semi.at[b])

    def wait_idx(b):
        pltpu.make_async_copy(src_hbm.at[pl.ds(0, K)], sidx.at[b, 0],
                              semi.at[b]).wait()
        pltpu.make_async_copy(dst_hbm.at[pl.ds(0, K)], didx.at[b, 0],
                              semi.at[b]).wait()
        pltpu.make_async_copy(e_hbm.at[pl.ds(0, K * H)], ebuf.at[b, 0],
                              semi.at[b]).wait()

    def issue_gather(b):
        pltpu.async_copy(hid_hbm.at[sidx.at[b, 0]], rows.at[b], semg.at[b])

    def wait_gather(b):
        pltpu.make_async_copy(hid_hbm.at[sidx.at[b, 0]], rows.at[b],
                              semg.at[b]).wait()

    def issue_scatter(rb):
        pltpu.async_copy(orows.at[rb], accm.at[didx_s.at[rb, 0]], sems.at[rb],
                         add=True)

    def wait_scatter(rb):
        pltpu.make_async_copy(orows.at[rb], accm.at[didx_s.at[rb, 0]],
                              sems.at[rb]).wait()

    bidx = [jnp.full((L, 1), v, jnp.int32) for v in range(L)]
    gdn = lax.GatherDimensionNumbers(
        offset_dims=(), collapsed_slice_dims=(0,), start_index_map=(0,))

    def compute(rb):
        for q in range(K // H):
            e16 = ebuf[rb, 0, pl.ds(q * L, L)]
            for j4 in range(H):
                k = q * H + j4
                for h in range(H):
                    ej = lax.gather(
                        e16, bidx[j4 * H + h], gdn, slice_sizes=(1,),
                        mode=lax.GatherScatterMode.PROMISE_IN_BOUNDS)
                    orows[rb, k, pl.ds(h * DH, L)] = rows[rb, k, pl.ds(h * DH, L)] * ej
                    orows[rb, k, pl.ds(h * DH + L, L)] = rows[rb, k, pl.ds(h * DH + L, L)] * ej
        for g in range(K // L):
            didx_s[rb, 0, pl.ds(g * L, L)] = didx[rb, 0, pl.ds(g * L, L)]

    issue_idx(0, 0)
    wait_idx(0)
    issue_gather(0)
    issue_idx(1, 1)

    NCH = EPT // K

    @pl.loop(0, NCH - 3, step=2)
    def _(c0):
        for j in range(2):
            c = c0 + j
            rb = j
            wait_gather(rb)
            wait_idx(1 - rb)
            issue_gather(1 - rb)

            @pl.when(c0 > 0)
            def _():
                wait_scatter(rb)

            compute(rb)
            issue_scatter(rb)
            issue_idx(c + 2, rb)

    for c in (NCH - 3, NCH - 2, NCH - 1):
        rb = c % 2
        wait_gather(rb)
        wait_scatter(rb)
        compute(rb)
        issue_scatter(rb)
        if c + 1 <= NCH - 1:
            wait_idx(1 - rb)
            issue_gather(1 - rb)
        if c + 2 <= NCH - 1:
            issue_idx(c + 2, rb)
    wait_scatter((NCH - 2) % 2)
    wait_scatter((NCH - 1) % 2)

    plsc.subcore_barrier()

    pltpu.sync_copy(accm.at[pl.ds(sid * RPT, RPT)],
                    outm_hbm.at[cid, pl.ds(sid * RPT, RPT)])

    @pl.when(sid == NS - 1)
    def _():
        pltpu.sync_copy(accm.at[pl.ds(NS * RPT, N - NS * RPT)],
                        outm_hbm.at[cid, pl.ds(NS * RPT, N - NS * RPT)])


@jax.jit
def _msg(hidden, e_all, src, dst):
    mesh = plsc.VectorSubcoreMesh(
        core_axis_name="c", subcore_axis_name="s",
        num_cores=NC, num_subcores=NS)
    cp = pltpu.CompilerParams()
    if "needs_layout_passes" in pltpu.CompilerParams.__dataclass_fields__:
        cp = dataclasses.replace(cp, needs_layout_passes=False)
    f = pl.kernel(
        _msg_body,
        out_type=jax.ShapeDtypeStruct((NC, N, D), jnp.float32),
        mesh=mesh,
        scratch_types=[
            pltpu.VMEM((2, 1, K), jnp.int32),
            pltpu.VMEM((2, 1, K), jnp.int32),
            pltpu.VMEM((2, 1, K), jnp.int32),
            pltpu.VMEM((2, 1, K * H), jnp.float32),
            pltpu.VMEM((2, K, D), jnp.float32),
            pltpu.VMEM((2, K, D), jnp.float32),
            pltpu.VMEM_SHARED((N, D), jnp.float32),
            pltpu.SemaphoreType.DMA((2,)),
            pltpu.SemaphoreType.DMA((2,)),
            pltpu.SemaphoreType.DMA((2,)),
        ],
        compiler_params=cp,
    )
    return f(hidden, e_all, src, dst)


def _finish_body(pa_ref, pb_ref, ea_ref, eb_ref, hid_ref, a_ref, n2g_ref,
                 nf_ref, hg_ref, gsum, gcnt):
    i = pl.program_id(0)
    msg = pa_ref[...] + pb_ref[...]
    esum = ea_ref[...] + eb_ref[...]
    hid = hid_ref[...]
    a = a_ref[...]

    ws = a[:, :H] + a[:, H:]
    ws = jnp.where(ws >= 0.0, ws, NEG * ws)
    es = jnp.exp(ws)

    parts = []
    for h in range(H):
        msg_h = (msg[:, h * DH:(h + 1) * DH]
                 + es[:, h:h + 1] * hid[:, h * DH:(h + 1) * DH])
        den_h = esum[:, h:h + 1] + es[:, h:h + 1]
        parts.append(msg_h / den_h)
    nf = jnp.maximum(jnp.concatenate(parts, axis=1), 0.0)
    nf_ref[...] = nf

    n2g = n2g_ref[...]
    iota = lax.broadcasted_iota(jnp.int32, (BN, G), 1).astype(jnp.float32)
    oh = (n2g == iota).astype(jnp.float32)
    dn = (((0,), (0,)), ((), ()))
    blk_sum = lax.dot_general(oh, nf, dn, preferred_element_type=jnp.float32)
    blk_cnt = lax.dot_general(oh, jnp.ones((BN, 1), jnp.float32), dn,
                              preferred_element_type=jnp.float32)

    @pl.when(i == 0)
    def _():
        gsum[...] = jnp.zeros_like(gsum)
        gcnt[...] = jnp.zeros_like(gcnt)

    gsum[...] += blk_sum
    gcnt[...] += blk_cnt

    @pl.when(i == NB - 1)
    def _():
        hg_ref[...] = gsum[...] / jnp.maximum(gcnt[...], 1.0)


@jax.jit
def _finish(pa, pb, ea, eb, hidden, a, n2g_f):
    return pl.pallas_call(
        _finish_body,
        grid=(NB,),
        in_specs=[
            pl.BlockSpec((BN, D), lambda i: (i, 0)),
            pl.BlockSpec((BN, D), lambda i: (i, 0)),
            pl.BlockSpec((BN, H), lambda i: (i, 0)),
            pl.BlockSpec((BN, H), lambda i: (i, 0)),
            pl.BlockSpec((BN, D), lambda i: (i, 0)),
            pl.BlockSpec((BN, 2 * H), lambda i: (i, 0)),
            pl.BlockSpec((BN, 1), lambda i: (i, 0)),
        ],
        out_specs=[
            pl.BlockSpec((BN, D), lambda i: (i, 0)),
            pl.BlockSpec((G, D), lambda i: (0, 0)),
        ],
        out_shape=[
            jax.ShapeDtypeStruct((N, D), jnp.float32),
            jax.ShapeDtypeStruct((G, D), jnp.float32),
        ],
        scratch_shapes=[
            pltpu.VMEM((G, D), jnp.float32),
            pltpu.VMEM((G, 1), jnp.float32),
        ],
    )(pa, pb, ea, eb, hidden, a, n2g_f)


def kernel(node_feature, edge_index, node2graph, W, b, query):
    qi = query[:, 0::2]
    qo = query[:, 1::2]
    Q = jnp.zeros((D, 2 * H), jnp.float32)
    for h in range(H):
        Q = Q.at[h * DH:(h + 1) * DH, h].set(qi[h])
        Q = Q.at[h * DH:(h + 1) * DH, H + h].set(qo[h])

    hidden, a = _prep(node_feature, W.T, b.reshape(1, D), Q)
    src, dst = edge_index[0], edge_index[1]
    e_all, e2 = _att(a.reshape(-1), src, dst)
    msg2 = _msg(hidden, e_all, src, dst)
    ea = e2[0].reshape(-1)[:N * H].reshape(N, H)
    eb = e2[1].reshape(-1)[:N * H].reshape(N, H)
    n2g_f = node2graph.astype(jnp.float32).reshape(N, 1)
    new_nf, hg = _finish(msg2[0], msg2[1], ea, eb, hidden, a, n2g_f)
    return new_nf, hg
